# baseline (speedup 1.0000x reference)
import functools

import jax
import jax.numpy as jnp
from jax import lax
from jax.experimental import pallas as pl
from jax.experimental.pallas import tpu as pltpu
from jax.experimental.pallas import tpu_sc as plsc

N = 10000
K = 16
M = N * K
TN = 200
TM = TN * K
GRID = M // TM
CNT = float(M)
EPS = 1e-5

_SC_CHUNK = 200


def _sc_gather(nf_t, idx1, idx0):
    info = plsc.get_sparse_core_info()
    nc, ns = info.num_cores, info.num_subcores
    nw = nc * ns
    per_w = M // nw
    n_it = per_w // _SC_CHUNK
    mesh = plsc.VectorSubcoreMesh(core_axis_name="c", subcore_axis_name="s")

    @functools.partial(
        pl.kernel,
        mesh=mesh,
        compiler_params=pltpu.CompilerParams(use_tc_tiling_on_sc=False),
        out_type=(
            jax.ShapeDtypeStruct((M, 64), jnp.int32),
            jax.ShapeDtypeStruct((M, 64), jnp.int32),
        ),
        scratch_types=[
            pltpu.VMEM((_SC_CHUNK,), jnp.int32),
            pltpu.VMEM((_SC_CHUNK, 64), jnp.int32),
            pltpu.VMEM((_SC_CHUNK,), jnp.int32),
            pltpu.VMEM((_SC_CHUNK, 64), jnp.int32),
            pltpu.SemaphoreType.DMA,
            pltpu.SemaphoreType.DMA,
        ],
    )
    def k(nf_hbm, i1_hbm, i0_hbm, hi_hbm, hj_hbm,
          idx_a, rows_a, idx_b, rows_b, sem_a, sem_b):
        wid = lax.axis_index("s") * nc + lax.axis_index("c")
        base0 = wid * per_w

        def body(c, _):
            base = base0 + c * _SC_CHUNK
            pltpu.sync_copy(i1_hbm.at[pl.ds(base, _SC_CHUNK)], idx_a)
            cp_a = pltpu.async_copy(nf_hbm.at[idx_a], rows_a, sem_a)
            pltpu.sync_copy(i0_hbm.at[pl.ds(base, _SC_CHUNK)], idx_b)
            cp_b = pltpu.async_copy(nf_hbm.at[idx_b], rows_b, sem_b)
            cp_a.wait()
            pltpu.sync_copy(rows_a, hi_hbm.at[pl.ds(base, _SC_CHUNK)])
            cp_b.wait()
            pltpu.sync_copy(rows_b, hj_hbm.at[pl.ds(base, _SC_CHUNK)])
            return ()

        lax.fori_loop(0, n_it, body, (), unroll=False)

    return k(nf_t, idx1, idx0)


def _bn_affine(sq, g, be):
    mean = sq[0:1, :] / CNT
    var = sq[1:2, :] / CNT - mean * mean
    a = g * lax.rsqrt(var + EPS)
    c = be - mean * a
    return a, c


def _acc_stats(i, y, acc_s, acc_q, sq_ref):
    s = jnp.sum(y, axis=0, keepdims=True)
    q = jnp.sum(y * y, axis=0, keepdims=True)

    @pl.when(i == 0)
    def _():
        acc_s[...] = jnp.zeros_like(acc_s)
        acc_q[...] = jnp.zeros_like(acc_q)

    acc_s[...] += s
    acc_q[...] += q

    @pl.when(i == GRID - 1)
    def _():
        sq_ref[0:1, :] = acc_s[...]
        sq_ref[1:2, :] = acc_q[...]


_TDIMS = (((1,), (1,)), ((), ()))


def _packed_matmul(p_i32, w_even, w_odd):
    even = lax.bitcast_convert_type(p_i32 << 16, jnp.float32)
    odd = lax.bitcast_convert_type(p_i32 & jnp.int32(-65536), jnp.float32)
    y = jnp.dot(even.astype(jnp.bfloat16), w_even[...],
                preferred_element_type=jnp.float32)
    y += jnp.dot(odd.astype(jnp.bfloat16), w_odd[...],
                 preferred_element_type=jnp.float32)
    return y


def _stage1_body(e_ref, hi_ref, hj_ref, w1ie, w1io, w1je, w1jo, w1e,
                 y1_ref, sq_ref, acc_s, acc_q):
    i = pl.program_id(0)
    y = lax.dot_general(e_ref[...], w1e[...], (((0,), (0,)), ((), ())),
                        preferred_element_type=jnp.float32,
                        precision=lax.Precision.HIGHEST)
    y += _packed_matmul(hi_ref[...], w1ie, w1io)
    y += _packed_matmul(hj_ref[...], w1je, w1jo)
    y1_ref[...] = y.astype(jnp.bfloat16)
    _acc_stats(i, y, acc_s, acc_q, sq_ref)


def _stage2_body(y1_ref, sq1_ref, g1, be1, w2,
                 y2_ref, sq_ref, acc_s, acc_q):
    i = pl.program_id(0)
    a, c = _bn_affine(sq1_ref[...], g1[...], be1[...])
    e1 = jnp.maximum(y1_ref[...].astype(jnp.float32) * a + c, 0.0)
    y = jnp.dot(e1.astype(jnp.bfloat16), w2[...],
                preferred_element_type=jnp.float32)
    y2_ref[...] = y.astype(jnp.bfloat16)
    _acc_stats(i, y, acc_s, acc_q, sq_ref)


def _stage3_body(y2_ref, sq2_ref, g2, be2, hi_ref, w3he, w3ho, w3m, eye,
                 e2t_ref, y3_ref, sq_ref, acc_s, acc_q):
    i = pl.program_id(0)
    a, c = _bn_affine(sq2_ref[...], g2[...], be2[...])
    e2 = jnp.maximum(y2_ref[...].astype(jnp.float32) * a + c, 0.0)
    e2t_ref[...] = lax.dot_general(eye[...], e2, _TDIMS,
                                   preferred_element_type=jnp.float32,
                                   precision=lax.Precision.HIGHEST)
    m = jnp.sum(e2.reshape(TN, K, 128), axis=1)
    mm = jnp.dot(m, w3m[...], preferred_element_type=jnp.float32,
                 precision=lax.Precision.HIGHEST)
    y = _packed_matmul(hi_ref[...], w3he, w3ho)
    y += jnp.broadcast_to(mm[:, None, :], (TN, K, 256)).reshape(TM, 256)
    y3_ref[...] = y
    _acc_stats(i, y, acc_s, acc_q, sq_ref)


def _stage4_body(y3_ref, sq3_ref, g3, be3, w4,
                 y4k0_ref, sq_ref, acc_s, acc_q):
    i = pl.program_id(0)
    a, c = _bn_affine(sq3_ref[...], g3[...], be3[...])
    n1 = jnp.maximum(y3_ref[...] * a + c, 0.0)
    y = jnp.dot(n1, w4[...], preferred_element_type=jnp.float32,
                precision=lax.Precision.HIGHEST)
    y4k0_ref[...] = y.reshape(TN, K, 128)[:, 0:1, :].reshape(TN, 128)
    _acc_stats(i, y, acc_s, acc_q, sq_ref)


def _stage5_body(y4_ref, sq4_ref, g4, be4, eye, out_ref):
    a, c = _bn_affine(sq4_ref[...], g4[...], be4[...])
    h = jnp.maximum(y4_ref[...] * a + c, 0.0)
    out_ref[...] = lax.dot_general(eye[...], h, _TDIMS,
                                   preferred_element_type=jnp.float32,
                                   precision=lax.Precision.HIGHEST)


def _row_spec(ch):
    return pl.BlockSpec((TM, ch), lambda i: (i, 0))


def _whole(shape):
    return pl.BlockSpec(shape, lambda i: tuple(0 for _ in shape))


def _sq_shape(ch):
    return jax.ShapeDtypeStruct((2, ch), jnp.float32)


def _scratch(ch):
    return [pltpu.VMEM((1, ch), jnp.float32),
            pltpu.VMEM((1, ch), jnp.float32)]


def _tc_chain(e_t, hi, hj, w1ie, w1io, w1je, w1jo, w1e, g1, be1, w2, g2, be2,
              w3he, w3ho, w3m, g3, be3, w4, g4, be4, eye):

    bf = jnp.bfloat16

    y1, sq1 = pl.pallas_call(
        _stage1_body,
        grid=(GRID,),
        in_specs=[pl.BlockSpec((16, TM), lambda i: (0, i)),
                  _row_spec(64), _row_spec(64),
                  _whole((64, 256)), _whole((64, 256)),
                  _whole((64, 256)), _whole((64, 256)), _whole((16, 256))],
        out_specs=[_row_spec(256), _whole((2, 256))],
        out_shape=[jax.ShapeDtypeStruct((M, 256), bf), _sq_shape(256)],
        scratch_shapes=_scratch(256),
    )(e_t, hi, hj, w1ie, w1io, w1je, w1jo, w1e)

    y2, sq2 = pl.pallas_call(
        _stage2_body,
        grid=(GRID,),
        in_specs=[_row_spec(256), _whole((2, 256)),
                  _whole((1, 256)), _whole((1, 256)), _whole((256, 128))],
        out_specs=[_row_spec(128), _whole((2, 128))],
        out_shape=[jax.ShapeDtypeStruct((M, 128), bf), _sq_shape(128)],
        scratch_shapes=_scratch(128),
    )(y1, sq1, g1, be1, w2)

    e2t, y3, sq3 = pl.pallas_call(
        _stage3_body,
        grid=(GRID,),
        in_specs=[_row_spec(128), _whole((2, 128)),
                  _whole((1, 128)), _whole((1, 128)), _row_spec(64),
                  _whole((64, 256)), _whole((64, 256)),
                  _whole((128, 256)), _whole((128, 128))],
        out_specs=[pl.BlockSpec((128, TM), lambda i: (0, i)),
                   _row_spec(256), _whole((2, 256))],
        out_shape=[jax.ShapeDtypeStruct((128, M), jnp.float32),
                   jax.ShapeDtypeStruct((M, 256), jnp.float32), _sq_shape(256)],
        scratch_shapes=_scratch(256),
    )(y2, sq2, g2, be2, hi, w3he, w3ho, w3m, eye)

    y4k0, sq4 = pl.pallas_call(
        _stage4_body,
        grid=(GRID,),
        in_specs=[_row_spec(256), _whole((2, 256)),
                  _whole((1, 256)), _whole((1, 256)), _whole((256, 128))],
        out_specs=[pl.BlockSpec((TN, 128), lambda i: (i, 0)),
                   _whole((2, 128))],
        out_shape=[jax.ShapeDtypeStruct((N, 128), jnp.float32), _sq_shape(128)],
        scratch_shapes=_scratch(128),
    )(y3, sq3, g3, be3, w4)

    h_out_t = pl.pallas_call(
        _stage5_body,
        in_specs=[pl.BlockSpec((N, 128), lambda: (0, 0)),
                  pl.BlockSpec((2, 128), lambda: (0, 0)),
                  pl.BlockSpec((1, 128), lambda: (0, 0)),
                  pl.BlockSpec((1, 128), lambda: (0, 0)),
                  pl.BlockSpec((128, 128), lambda: (0, 0))],
        out_specs=pl.BlockSpec((128, N), lambda: (0, 0)),
        out_shape=jax.ShapeDtypeStruct((128, N), jnp.float32),
    )(y4k0, sq4, g4, be4, eye)

    return e2t, h_out_t


def kernel(node_features, e_ij, edge_index,
           W1, b1, g1, be1, W2, b2, g2, be2,
           W3, b3, g3, be3, W4, b4, g4, be4):
    del b1, b2, b3, b4
    bf = jnp.bfloat16
    nf_bf = node_features[0, :, :, 0].T.astype(bf)
    nf_pk = lax.bitcast_convert_type(nf_bf.reshape(N, 64, 2), jnp.int32)
    e_cm = e_ij[0].reshape(16, M)
    idx1 = edge_index[1, 0].reshape(M)
    idx0 = edge_index[0, 0].reshape(M)

    hi, hj = _sc_gather(nf_pk, idx1, idx0)

    w1e = W1[:, :16].T
    w1i = W1[:, 16:144].T.astype(bf)
    w1j = W1[:, 144:].T.astype(bf)
    w3h = W3[:, :128].T.astype(bf)
    w3m = W3[:, 128:].T
    eye = jnp.eye(128, dtype=jnp.float32)
    r = lambda v: v.reshape(1, -1)

    e2t, h_out_t = _tc_chain(
        e_cm, hi, hj, w1i[0::2], w1i[1::2], w1j[0::2], w1j[1::2], w1e,
        r(g1), r(be1), W2.T.astype(bf),
        r(g2), r(be2), w3h[0::2], w3h[1::2], w3m, r(g3), r(be3), W4.T,
        r(g4), r(be4), eye)

    e_ij_prima = e2t.reshape(128, N, K)[None]
    h_i_prima = h_out_t[None, :, :, None]
    return (h_i_prima, e_ij_prima, edge_index)


# speedup vs baseline: 1.4529x; 1.4529x over previous
import functools

import jax
import jax.numpy as jnp
from jax import lax
from jax.experimental import pallas as pl
from jax.experimental.pallas import tpu as pltpu
from jax.experimental.pallas import tpu_sc as plsc

N = 10000
K = 16
M = N * K
TN = 200
TM = TN * K
GRID = M // TM
CNT = float(M)
EPS = 1e-5

_SC_CHUNK = 200


def _sc_gather(nf_t, idx1, idx0):
    info = plsc.get_sparse_core_info()
    nc, ns = info.num_cores, info.num_subcores
    nw = nc * ns
    per_w = M // nw
    n_it = per_w // _SC_CHUNK
    mesh = plsc.VectorSubcoreMesh(core_axis_name="c", subcore_axis_name="s")

    @functools.partial(
        pl.kernel,
        mesh=mesh,
        out_type=(
            jax.ShapeDtypeStruct((M, 128), jnp.float32),
            jax.ShapeDtypeStruct((M, 128), jnp.float32),
        ),
        scratch_types=[
            pltpu.VMEM((_SC_CHUNK,), jnp.int32),
            pltpu.VMEM((_SC_CHUNK, 128), jnp.float32),
            pltpu.VMEM((_SC_CHUNK,), jnp.int32),
            pltpu.VMEM((_SC_CHUNK, 128), jnp.float32),
            pltpu.SemaphoreType.DMA,
            pltpu.SemaphoreType.DMA,
        ],
    )
    def k(nf_hbm, i1_hbm, i0_hbm, hi_hbm, hj_hbm,
          idx_a, rows_a, idx_b, rows_b, sem_a, sem_b):
        wid = lax.axis_index("s") * nc + lax.axis_index("c")
        base0 = wid * per_w

        def body(c, _):
            base = base0 + c * _SC_CHUNK
            pltpu.sync_copy(i1_hbm.at[pl.ds(base, _SC_CHUNK)], idx_a)
            cp_a = pltpu.async_copy(nf_hbm.at[idx_a], rows_a, sem_a)
            pltpu.sync_copy(i0_hbm.at[pl.ds(base, _SC_CHUNK)], idx_b)
            cp_b = pltpu.async_copy(nf_hbm.at[idx_b], rows_b, sem_b)
            cp_a.wait()
            pltpu.sync_copy(rows_a, hi_hbm.at[pl.ds(base, _SC_CHUNK)])
            cp_b.wait()
            pltpu.sync_copy(rows_b, hj_hbm.at[pl.ds(base, _SC_CHUNK)])
            return ()

        lax.fori_loop(0, n_it, body, (), unroll=False)

    return k(nf_t, idx1, idx0)


def _bn_affine(sq, g, be):
    mean = sq[0:1, :] / CNT
    var = sq[1:2, :] / CNT - mean * mean
    a = g * lax.rsqrt(var + EPS)
    c = be - mean * a
    return a, c


def _acc_stats(i, y, acc_s, acc_q, sq_ref):
    s = jnp.sum(y, axis=0, keepdims=True)
    q = jnp.sum(y * y, axis=0, keepdims=True)

    @pl.when(i == 0)
    def _():
        acc_s[...] = jnp.zeros_like(acc_s)
        acc_q[...] = jnp.zeros_like(acc_q)

    acc_s[...] += s
    acc_q[...] += q

    @pl.when(i == GRID - 1)
    def _():
        sq_ref[0:1, :] = acc_s[...]
        sq_ref[1:2, :] = acc_q[...]


_TDIMS = (((1,), (1,)), ((), ()))


def _stage1_body(e_ref, hi_ref, hj_ref, w1i, w1j, w1e,
                 y1_ref, sq_ref, acc_s, acc_q):
    i = pl.program_id(0)
    y = lax.dot_general(e_ref[...], w1e[...], (((0,), (0,)), ((), ())),
                        preferred_element_type=jnp.float32)
    y += jnp.dot(hi_ref[...], w1i[...], preferred_element_type=jnp.float32)
    y += jnp.dot(hj_ref[...], w1j[...], preferred_element_type=jnp.float32)
    y1_ref[...] = y
    _acc_stats(i, y, acc_s, acc_q, sq_ref)


def _stage2_body(y1_ref, sq1_ref, g1, be1, w2,
                 y2_ref, sq_ref, acc_s, acc_q):
    i = pl.program_id(0)
    a, c = _bn_affine(sq1_ref[...], g1[...], be1[...])
    e1 = jnp.maximum(y1_ref[...] * a + c, 0.0)
    y = jnp.dot(e1, w2[...], preferred_element_type=jnp.float32)
    y2_ref[...] = y
    _acc_stats(i, y, acc_s, acc_q, sq_ref)


def _edge_y3(hi_ref, w3h, mm):
    y = jnp.dot(hi_ref[...], w3h[...], preferred_element_type=jnp.float32)
    return y + jnp.broadcast_to(mm[:, None, :], (TN, K, 256)).reshape(TM, 256)


def _stage3_body(y2_ref, sq2_ref, g2, be2, hi_ref, w3h, w3m, eye,
                 e2t_ref, mm_ref, sq_ref, acc_s, acc_q):
    i = pl.program_id(0)
    a, c = _bn_affine(sq2_ref[...], g2[...], be2[...])
    e2 = jnp.maximum(y2_ref[...] * a + c, 0.0)
    e2t_ref[...] = lax.dot_general(eye[...], e2, _TDIMS,
                                   preferred_element_type=jnp.float32)
    m = jnp.sum(e2.reshape(TN, K, 128), axis=1)
    mm = jnp.dot(m, w3m[...], preferred_element_type=jnp.float32)
    mm_ref[...] = mm
    y = _edge_y3(hi_ref, w3h, mm)
    _acc_stats(i, y, acc_s, acc_q, sq_ref)


def _stage4_body(mm_ref, sq3_ref, g3, be3, hi_ref, w3h, w4,
                 y4k0_ref, sq_ref, acc_s, acc_q):
    i = pl.program_id(0)
    a, c = _bn_affine(sq3_ref[...], g3[...], be3[...])
    y3 = _edge_y3(hi_ref, w3h, mm_ref[...])
    n1 = jnp.maximum(y3 * a + c, 0.0)
    y = jnp.dot(n1, w4[...], preferred_element_type=jnp.float32)
    y4k0_ref[...] = y.reshape(TN, K, 128)[:, 0:1, :].reshape(TN, 128)
    _acc_stats(i, y, acc_s, acc_q, sq_ref)


def _stage5_body(y4_ref, sq4_ref, g4, be4, eye, out_ref):
    a, c = _bn_affine(sq4_ref[...], g4[...], be4[...])
    h = jnp.maximum(y4_ref[...] * a + c, 0.0)
    out_ref[...] = lax.dot_general(eye[...], h, _TDIMS,
                                   preferred_element_type=jnp.float32)


def _row_spec(ch):
    return pl.BlockSpec((TM, ch), lambda i: (i, 0))


def _whole(shape):
    return pl.BlockSpec(shape, lambda i: tuple(0 for _ in shape))


def _sq_shape(ch):
    return jax.ShapeDtypeStruct((2, ch), jnp.float32)


def _scratch(ch):
    return [pltpu.VMEM((1, ch), jnp.float32),
            pltpu.VMEM((1, ch), jnp.float32)]


def _tc_chain(e_t, hi, hj, w1i, w1j, w1e, g1, be1, w2, g2, be2,
              w3h, w3m, g3, be3, w4, g4, be4, eye):

    y1, sq1 = pl.pallas_call(
        _stage1_body,
        grid=(GRID,),
        in_specs=[pl.BlockSpec((16, TM), lambda i: (0, i)),
                  _row_spec(128), _row_spec(128),
                  _whole((128, 256)), _whole((128, 256)), _whole((16, 256))],
        out_specs=[_row_spec(256), _whole((2, 256))],
        out_shape=[jax.ShapeDtypeStruct((M, 256), jnp.float32), _sq_shape(256)],
        scratch_shapes=_scratch(256),
    )(e_t, hi, hj, w1i, w1j, w1e)

    y2, sq2 = pl.pallas_call(
        _stage2_body,
        grid=(GRID,),
        in_specs=[_row_spec(256), _whole((2, 256)),
                  _whole((1, 256)), _whole((1, 256)), _whole((256, 128))],
        out_specs=[_row_spec(128), _whole((2, 128))],
        out_shape=[jax.ShapeDtypeStruct((M, 128), jnp.float32), _sq_shape(128)],
        scratch_shapes=_scratch(128),
    )(y1, sq1, g1, be1, w2)

    e2t, mm, sq3 = pl.pallas_call(
        _stage3_body,
        grid=(GRID,),
        in_specs=[_row_spec(128), _whole((2, 128)),
                  _whole((1, 128)), _whole((1, 128)), _row_spec(128),
                  _whole((128, 256)), _whole((128, 256)), _whole((128, 128))],
        out_specs=[pl.BlockSpec((128, TM), lambda i: (0, i)),
                   pl.BlockSpec((TN, 256), lambda i: (i, 0)),
                   _whole((2, 256))],
        out_shape=[jax.ShapeDtypeStruct((128, M), jnp.float32),
                   jax.ShapeDtypeStruct((N, 256), jnp.float32), _sq_shape(256)],
        scratch_shapes=_scratch(256),
    )(y2, sq2, g2, be2, hi, w3h, w3m, eye)

    y4k0, sq4 = pl.pallas_call(
        _stage4_body,
        grid=(GRID,),
        in_specs=[pl.BlockSpec((TN, 256), lambda i: (i, 0)), _whole((2, 256)),
                  _whole((1, 256)), _whole((1, 256)), _row_spec(128),
                  _whole((128, 256)), _whole((256, 128))],
        out_specs=[pl.BlockSpec((TN, 128), lambda i: (i, 0)),
                   _whole((2, 128))],
        out_shape=[jax.ShapeDtypeStruct((N, 128), jnp.float32), _sq_shape(128)],
        scratch_shapes=_scratch(128),
    )(mm, sq3, g3, be3, hi, w3h, w4)

    h_out_t = pl.pallas_call(
        _stage5_body,
        in_specs=[pl.BlockSpec((N, 128), lambda: (0, 0)),
                  pl.BlockSpec((2, 128), lambda: (0, 0)),
                  pl.BlockSpec((1, 128), lambda: (0, 0)),
                  pl.BlockSpec((1, 128), lambda: (0, 0)),
                  pl.BlockSpec((128, 128), lambda: (0, 0))],
        out_specs=pl.BlockSpec((128, N), lambda: (0, 0)),
        out_shape=jax.ShapeDtypeStruct((128, N), jnp.float32),
    )(y4k0, sq4, g4, be4, eye)

    return e2t, h_out_t


def kernel(node_features, e_ij, edge_index,
           W1, b1, g1, be1, W2, b2, g2, be2,
           W3, b3, g3, be3, W4, b4, g4, be4):
    del b1, b2, b3, b4
    nf_t = node_features[0, :, :, 0].T
    e_cm = e_ij[0].reshape(16, M)
    idx1 = edge_index[1, 0].reshape(M)
    idx0 = edge_index[0, 0].reshape(M)

    hi, hj = _sc_gather(nf_t, idx1, idx0)

    w1e = W1[:, :16].T
    w1i = W1[:, 16:144].T
    w1j = W1[:, 144:].T
    w3h = W3[:, :128].T
    w3m = W3[:, 128:].T
    eye = jnp.eye(128, dtype=jnp.float32)
    r = lambda v: v.reshape(1, -1)

    e2t, h_out_t = _tc_chain(
        e_cm, hi, hj, w1i, w1j, w1e, r(g1), r(be1), W2.T,
        r(g2), r(be2), w3h, w3m, r(g3), r(be3), W4.T,
        r(g4), r(be4), eye)

    e_ij_prima = e2t.reshape(128, N, K)[None]
    h_i_prima = h_out_t[None, :, :, None]
    return (h_i_prima, e_ij_prima, edge_index)
